# baseline (speedup 1.0000x reference)
import functools

import jax
import jax.numpy as jnp
from jax import lax
from jax.experimental import pallas as pl
from jax.experimental.pallas import tpu as pltpu
from jax.experimental.pallas import tpu_sc as plsc

_SC_PARAMS = pltpu.CompilerParams(use_tc_tiling_on_sc=False,
                                  needs_layout_passes=False)

N = 10000
E = 320000
NR = 8
NBINS = NR * N
NSUB = 16
NCORE = 2
LANE = 16
K = 400
RING = 5
ROWS_PER_SUB = NBINS // NSUB


def _tables_tc(h, W_rel):
    Din = h.shape[1]
    D = W_rel.shape[2]
    NCH = D // LANE
    wt = W_rel.reshape(NR, Din, NCH, LANE).transpose(2, 0, 1, 3)
    wt = wt.astype(jnp.bfloat16)
    eye8 = jnp.eye(8, dtype=jnp.bfloat16)
    W8 = (eye8[None, None, :, None, :, None]
          * wt[:, :, None, :, None, :]).reshape(NCH, NR, 8 * Din, 8 * LANE)
    x8 = h.reshape(N // 8, 8 * Din).astype(jnp.bfloat16)

    def body(x_ref, w_ref, o_ref):
        o_ref[0] = jnp.dot(x_ref[...], w_ref[0, 0],
                           preferred_element_type=jnp.float32)

    out = pl.pallas_call(
        body,
        grid=(NCH, NR),
        in_specs=[
            pl.BlockSpec((N // 8, 8 * Din), lambda c, r: (0, 0)),
            pl.BlockSpec((1, 1, 8 * Din, 8 * LANE),
                         lambda c, r: (c, r, 0, 0)),
        ],
        out_specs=pl.BlockSpec((1, N // 8, 8 * LANE),
                               lambda c, r: (c * NR + r, 0, 0)),
        out_shape=jax.ShapeDtypeStruct((NCH * NR, N // 8, 8 * LANE),
                                       jnp.float32),
    )(x8, W8)
    return out.reshape(NCH * NR * N, LANE)


def _counts_sc(pk):
    eps = E // (NCORE * NSUB)
    nb = eps // K
    mesh = plsc.VectorSubcoreMesh(core_axis_name="c", subcore_axis_name="s")

    @functools.partial(
        pl.kernel,
        out_type=jax.ShapeDtypeStruct((NCORE * NBINS,), jnp.float32),
        mesh=mesh,
        compiler_params=_SC_PARAMS,
        scratch_types=[
            pltpu.VMEM((2, K), jnp.int32),
            pltpu.VMEM((K, LANE), jnp.float32),
            pltpu.VMEM((1264, LANE), jnp.float32),
            pltpu.VMEM((ROWS_PER_SUB,), jnp.float32),
            pltpu.VMEM_SHARED((NBINS, LANE), jnp.float32),
        ],
    )
    def k(pk_hbm, out_hbm, pk_v, ones_v, piece_v, cnt_v, accum):
        core = lax.axis_index("c")
        sub = lax.axis_index("s")

        @pl.loop(0, K)
        def _(i):
            ones_v[i, :] = jnp.full((LANE,), 1.0, jnp.float32)

        @pl.loop(0, 1250)
        def _(i):
            piece_v[i, :] = jnp.zeros((LANE,), jnp.float32)

        @pl.loop(0, 4)
        def _(i):
            pltpu.sync_copy(piece_v.at[pl.ds(0, 1250)],
                            accum.at[pl.ds(sub * ROWS_PER_SUB + i * 1250, 1250)])
        plsc.subcore_barrier()

        mbase = (core * NSUB + sub) * nb

        @pl.loop(0, nb)
        def _(b):
            pltpu.sync_copy(pk_hbm.at[mbase + b], pk_v)
            pltpu.sync_copy(ones_v, accum.at[pk_v.at[1]], add=True)
        plsc.subcore_barrier()

        @pl.loop(0, 4)
        def _(i):
            start = jnp.minimum(i * 1250, ROWS_PER_SUB - 1264)
            pltpu.sync_copy(accum.at[pl.ds(sub * ROWS_PER_SUB + start, 1264)],
                            piece_v)

            @pl.loop(0, 1264 // LANE)
            def _(q):
                rows = q * LANE + lax.iota(jnp.int32, LANE)
                vals = plsc.load_gather(piece_v,
                                        [rows, jnp.zeros((LANE,), jnp.int32)])
                cnt_v[pl.ds(start + q * LANE, LANE)] = vals

        pltpu.sync_copy(
            cnt_v,
            out_hbm.at[pl.ds(core * NBINS + sub * ROWS_PER_SUB,
                             ROWS_PER_SUB)])

    return k(pk)


def _scatter_sc(table, pk, nch):
    cpc = nch // NCORE
    eps = E // NSUB
    nb = eps // K
    mesh = plsc.VectorSubcoreMesh(core_axis_name="c", subcore_axis_name="s")

    @functools.partial(
        pl.kernel,
        out_type=jax.ShapeDtypeStruct((NR, N, 8 * LANE), jnp.float32),
        mesh=mesh,
        compiler_params=_SC_PARAMS,
        scratch_types=[
            pltpu.VMEM((RING, 2, K), jnp.int32),
            pltpu.VMEM((RING, K), jnp.int32),
            pltpu.VMEM((RING, K, LANE), jnp.float32),
            pltpu.VMEM((625, LANE), jnp.float32),
            pltpu.VMEM_SHARED((NBINS, LANE), jnp.float32),
        ] + [pltpu.SemaphoreType.DMA] * (2 * RING),
    )
    def k(table_hbm, pk_hbm, out_hbm,
          pk_v, idx_v, rows_v, zero_v, accum, *sems):
        sem_g = sems[:RING]
        sem_s = sems[RING:]
        core = lax.axis_index("c")
        sub = lax.axis_index("s")
        mbase = sub * nb
        r0 = sub // 2
        n0 = (sub % 2) * ROWS_PER_SUB

        @pl.loop(0, 625)
        def _(i):
            zero_v[i, :] = jnp.zeros((LANE,), jnp.float32)

        def load_batch(j, b, off):
            pltpu.sync_copy(pk_hbm.at[mbase + b], pk_v.at[j])

            @pl.loop(0, K // LANE)
            def _(i):
                sl = pl.ds(i * LANE, LANE)
                idx_v[j, sl] = pk_v[j, 0, sl] + off

        def gather(j):
            pltpu.async_copy(table_hbm.at[idx_v.at[j]], rows_v.at[j],
                             sem_g[j])

        def wait_g(j):
            pltpu.make_async_copy(table_hbm.at[idx_v.at[j]], rows_v.at[j],
                                  sem_g[j]).wait()

        def scatter(j):
            pltpu.async_copy(rows_v.at[j], accum.at[pk_v.at[j, 1]],
                             sem_s[j], add=True)

        def wait_s(j):
            pltpu.make_async_copy(rows_v.at[j], accum.at[pk_v.at[j, 1]],
                                  sem_s[j]).wait()

        for kk in range(cpc):
            g = core * cpc + kk
            off = g * NBINS

            @pl.loop(0, 8)
            def _(i):
                pltpu.sync_copy(
                    zero_v,
                    accum.at[pl.ds(sub * ROWS_PER_SUB + i * 625, 625)])
            plsc.subcore_barrier()

            for j in range(RING):
                load_batch(j, j, off)
                gather(j)
            wait_g(0)
            scatter(0)

            @pl.loop(1, nb // RING)
            def _(p):
                for j in range(RING):
                    b = p * RING + j
                    wait_s(j)
                    load_batch(j, b, off)
                    gather(j)
                    jo = (j + 1) % RING
                    wait_g(jo)
                    scatter(jo)

            for j in range(1, RING):
                wait_g(j)
                scatter(j)
            for j in range(RING):
                wait_s(j)
            plsc.subcore_barrier()

            pltpu.sync_copy(
                accum.at[pl.ds(sub * ROWS_PER_SUB, ROWS_PER_SUB)],
                out_hbm.at[r0, pl.ds(n0, ROWS_PER_SUB),
                           pl.ds(g * LANE, LANE)])

    return k(table, pk)


def _combine_tc(h, W_root, b, acc, counts2, relu):
    D = W_root.shape[1]
    BN = 5000

    def body(x_ref, w_ref, b_ref, a_ref, c_ref, o_ref):
        r = pl.program_id(1)
        cnt = c_ref[0, 0, :, 0] + c_ref[1, 0, :, 0]
        inv = 1.0 / jnp.maximum(cnt, 1.0)
        contrib = a_ref[0, :, :D] * inv[:, None]

        @pl.when(r == 0)
        def _():
            o_ref[...] = jnp.dot(x_ref[...], w_ref[...],
                                 preferred_element_type=jnp.float32) \
                + b_ref[0] + contrib

        @pl.when(r > 0)
        def _():
            o_ref[...] += contrib

        if relu:
            @pl.when(r == NR - 1)
            def _():
                o_ref[...] = jnp.maximum(o_ref[...], 0.0)

    return pl.pallas_call(
        body,
        grid=(N // BN, NR),
        in_specs=[
            pl.BlockSpec((BN, h.shape[1]), lambda n, r: (n, 0)),
            pl.BlockSpec((h.shape[1], D), lambda n, r: (0, 0)),
            pl.BlockSpec((1, D), lambda n, r: (0, 0)),
            pl.BlockSpec((1, BN, 8 * LANE), lambda n, r: (r, n, 0)),
            pl.BlockSpec((NCORE, 1, BN, 1), lambda n, r: (0, r, n, 0)),
        ],
        out_specs=pl.BlockSpec((BN, D), lambda n, r: (n, 0)),
        out_shape=jax.ShapeDtypeStruct((N, D), jnp.float32),
    )(h, W_root, b.reshape(1, D), acc, counts2)


def kernel(x, edge_index, edge_type, W1_rel, W1_root, b1, W2_rel, W2_root, b2):
    ei = edge_index.astype(jnp.int32)
    et = edge_type.astype(jnp.int32)
    gidx = et * N + ei[0]
    sidx = et * N + ei[1]
    pk = jnp.stack([gidx.reshape(E // K, K), sidx.reshape(E // K, K)], axis=1)

    counts1d = _counts_sc(pk)
    counts2 = counts1d.reshape(NCORE, NR, N, 1)

    t1 = _tables_tc(x, W1_rel)
    a1 = _scatter_sc(t1, pk, W1_rel.shape[2] // LANE)
    h = _combine_tc(x, W1_root, b1, a1, counts2, relu=True)

    t2 = _tables_tc(h, W2_rel)
    a2 = _scatter_sc(t2, pk, W2_rel.shape[2] // LANE)
    out = _combine_tc(h, W2_root, b2, a2, counts2, relu=False)
    return out


# speedup vs baseline: 1.0864x; 1.0864x over previous
import functools

import jax
import jax.numpy as jnp
from jax import lax
from jax.experimental import pallas as pl
from jax.experimental.pallas import tpu as pltpu
from jax.experimental.pallas import tpu_sc as plsc

_SC_PARAMS = pltpu.CompilerParams(use_tc_tiling_on_sc=False,
                                  needs_layout_passes=False)

N = 10000
E = 320000
NR = 8
NBINS = NR * N
NSUB = 16
NCORE = 2
LANE = 16
K = 400
RING = 5
ROWS_PER_SUB = NBINS // NSUB


def _tables_tc(h, W_rel):
    Din = h.shape[1]
    D = W_rel.shape[2]
    NCH = D // LANE
    wt = W_rel.reshape(NR, Din, NCH, LANE).transpose(2, 0, 1, 3)
    wt = wt.astype(jnp.bfloat16)
    eye8 = jnp.eye(8, dtype=jnp.bfloat16)
    W8 = (eye8[None, None, :, None, :, None]
          * wt[:, :, None, :, None, :]).reshape(NCH, NR, 8 * Din, 8 * LANE)
    x8 = h.reshape(N // 8, 8 * Din).astype(jnp.bfloat16)

    W8 = W8.reshape(NCH * NR // 4, 4, 8 * Din, 8 * LANE)

    def body(x_ref, w_ref, o_ref):
        for i in range(4):
            y = jnp.dot(x_ref[...], w_ref[0, i],
                        preferred_element_type=jnp.float32)
            o_ref[pl.ds(i * N * LANE, N * LANE)] = y.reshape(N * LANE)

    out = pl.pallas_call(
        body,
        grid=(NCH * NR // 4,),
        in_specs=[
            pl.BlockSpec((N // 8, 8 * Din), lambda q: (0, 0)),
            pl.BlockSpec((1, 4, 8 * Din, 8 * LANE), lambda q: (q, 0, 0, 0)),
        ],
        out_specs=pl.BlockSpec((4 * N * LANE,), lambda q: (q,)),
        out_shape=jax.ShapeDtypeStruct((NCH * NR * N * LANE,), jnp.float32),
    )(x8, W8)
    return out.reshape(NCH * NR * N, LANE)


def _counts_sc(pk):
    eps = E // (NCORE * NSUB)
    nb = eps // K
    mesh = plsc.VectorSubcoreMesh(core_axis_name="c", subcore_axis_name="s")

    @functools.partial(
        pl.kernel,
        out_type=jax.ShapeDtypeStruct((NCORE * NBINS,), jnp.float32),
        mesh=mesh,
        compiler_params=_SC_PARAMS,
        scratch_types=[
            pltpu.VMEM((2, K), jnp.int32),
            pltpu.VMEM((K, LANE), jnp.float32),
            pltpu.VMEM((1264, LANE), jnp.float32),
            pltpu.VMEM((ROWS_PER_SUB,), jnp.float32),
            pltpu.VMEM_SHARED((NBINS, LANE), jnp.float32),
        ],
    )
    def k(pk_hbm, out_hbm, pk_v, ones_v, piece_v, cnt_v, accum):
        core = lax.axis_index("c")
        sub = lax.axis_index("s")

        @pl.loop(0, K)
        def _(i):
            ones_v[i, :] = jnp.full((LANE,), 1.0, jnp.float32)

        @pl.loop(0, 1250)
        def _(i):
            piece_v[i, :] = jnp.zeros((LANE,), jnp.float32)

        @pl.loop(0, 4)
        def _(i):
            pltpu.sync_copy(piece_v.at[pl.ds(0, 1250)],
                            accum.at[pl.ds(sub * ROWS_PER_SUB + i * 1250, 1250)])
        plsc.subcore_barrier()

        mbase = (core * NSUB + sub) * nb

        @pl.loop(0, nb)
        def _(b):
            pltpu.sync_copy(pk_hbm.at[mbase + b], pk_v)
            pltpu.sync_copy(ones_v, accum.at[pk_v.at[1]], add=True)
        plsc.subcore_barrier()

        @pl.loop(0, 4)
        def _(i):
            start = jnp.minimum(i * 1250, ROWS_PER_SUB - 1264)
            pltpu.sync_copy(accum.at[pl.ds(sub * ROWS_PER_SUB + start, 1264)],
                            piece_v)

            @pl.loop(0, 1264 // LANE)
            def _(q):
                rows = q * LANE + lax.iota(jnp.int32, LANE)
                vals = plsc.load_gather(piece_v,
                                        [rows, jnp.zeros((LANE,), jnp.int32)])
                cnt_v[pl.ds(start + q * LANE, LANE)] = vals

        pltpu.sync_copy(
            cnt_v,
            out_hbm.at[pl.ds(core * NBINS + sub * ROWS_PER_SUB,
                             ROWS_PER_SUB)])

    return k(pk)


def _scatter_sc(table, pk, nch):
    cpc = nch // NCORE
    eps = E // NSUB
    nb = eps // K
    mesh = plsc.VectorSubcoreMesh(core_axis_name="c", subcore_axis_name="s")

    @functools.partial(
        pl.kernel,
        out_type=jax.ShapeDtypeStruct((NR, N, 8 * LANE), jnp.float32),
        mesh=mesh,
        compiler_params=_SC_PARAMS,
        scratch_types=[
            pltpu.VMEM((RING, 2, K), jnp.int32),
            pltpu.VMEM((RING, K), jnp.int32),
            pltpu.VMEM((RING, K, LANE), jnp.float32),
            pltpu.VMEM((625, LANE), jnp.float32),
            pltpu.VMEM_SHARED((NBINS, LANE), jnp.float32),
        ] + [pltpu.SemaphoreType.DMA] * (2 * RING),
    )
    def k(table_hbm, pk_hbm, out_hbm,
          pk_v, idx_v, rows_v, zero_v, accum, *sems):
        sem_g = sems[:RING]
        sem_s = sems[RING:]
        core = lax.axis_index("c")
        sub = lax.axis_index("s")
        mbase = sub * nb
        r0 = sub // 2
        n0 = (sub % 2) * ROWS_PER_SUB

        @pl.loop(0, 625)
        def _(i):
            zero_v[i, :] = jnp.zeros((LANE,), jnp.float32)

        def load_batch(j, b, off):
            pltpu.sync_copy(pk_hbm.at[mbase + b], pk_v.at[j])

            @pl.loop(0, K // LANE)
            def _(i):
                sl = pl.ds(i * LANE, LANE)
                idx_v[j, sl] = pk_v[j, 0, sl] + off

        def gather(j):
            pltpu.async_copy(table_hbm.at[idx_v.at[j]], rows_v.at[j],
                             sem_g[j])

        def wait_g(j):
            pltpu.make_async_copy(table_hbm.at[idx_v.at[j]], rows_v.at[j],
                                  sem_g[j]).wait()

        def scatter(j):
            pltpu.async_copy(rows_v.at[j], accum.at[pk_v.at[j, 1]],
                             sem_s[j], add=True)

        def wait_s(j):
            pltpu.make_async_copy(rows_v.at[j], accum.at[pk_v.at[j, 1]],
                                  sem_s[j]).wait()

        for kk in range(cpc):
            g = core * cpc + kk
            off = g * NBINS

            @pl.loop(0, 8)
            def _(i):
                pltpu.sync_copy(
                    zero_v,
                    accum.at[pl.ds(sub * ROWS_PER_SUB + i * 625, 625)])
            plsc.subcore_barrier()

            for j in range(RING):
                load_batch(j, j, off)
                gather(j)
            wait_g(0)
            scatter(0)

            @pl.loop(1, nb // RING)
            def _(p):
                for j in range(RING):
                    b = p * RING + j
                    wait_s(j)
                    load_batch(j, b, off)
                    gather(j)
                    jo = (j + 1) % RING
                    wait_g(jo)
                    scatter(jo)

            for j in range(1, RING):
                wait_g(j)
                scatter(j)
            for j in range(RING):
                wait_s(j)
            plsc.subcore_barrier()

            pltpu.sync_copy(
                accum.at[pl.ds(sub * ROWS_PER_SUB, ROWS_PER_SUB)],
                out_hbm.at[r0, pl.ds(n0, ROWS_PER_SUB),
                           pl.ds(g * LANE, LANE)])

    return k(table, pk)


def _combine_tc(h, W_root, b, acc, counts2, relu):
    D = W_root.shape[1]
    BN = 5000

    def body(x_ref, w_ref, b_ref, a_ref, c_ref, o_ref):
        r = pl.program_id(1)
        cnt = c_ref[0, 0, :, 0] + c_ref[1, 0, :, 0]
        inv = 1.0 / jnp.maximum(cnt, 1.0)
        contrib = a_ref[0, :, :D] * inv[:, None]

        @pl.when(r == 0)
        def _():
            o_ref[...] = jnp.dot(x_ref[...], w_ref[...],
                                 preferred_element_type=jnp.float32) \
                + b_ref[0] + contrib

        @pl.when(r > 0)
        def _():
            o_ref[...] += contrib

        if relu:
            @pl.when(r == NR - 1)
            def _():
                o_ref[...] = jnp.maximum(o_ref[...], 0.0)

    return pl.pallas_call(
        body,
        grid=(N // BN, NR),
        in_specs=[
            pl.BlockSpec((BN, h.shape[1]), lambda n, r: (n, 0)),
            pl.BlockSpec((h.shape[1], D), lambda n, r: (0, 0)),
            pl.BlockSpec((1, D), lambda n, r: (0, 0)),
            pl.BlockSpec((1, BN, 8 * LANE), lambda n, r: (r, n, 0)),
            pl.BlockSpec((NCORE, 1, BN, 1), lambda n, r: (0, r, n, 0)),
        ],
        out_specs=pl.BlockSpec((BN, D), lambda n, r: (n, 0)),
        out_shape=jax.ShapeDtypeStruct((N, D), jnp.float32),
    )(h, W_root, b.reshape(1, D), acc, counts2)


def kernel(x, edge_index, edge_type, W1_rel, W1_root, b1, W2_rel, W2_root, b2):
    ei = edge_index.astype(jnp.int32)
    et = edge_type.astype(jnp.int32)
    gidx = et * N + ei[0]
    sidx = et * N + ei[1]
    pk = jnp.stack([gidx.reshape(E // K, K), sidx.reshape(E // K, K)], axis=1)

    counts1d = _counts_sc(pk)
    counts2 = counts1d.reshape(NCORE, NR, N, 1)

    t1 = _tables_tc(x, W1_rel)
    a1 = _scatter_sc(t1, pk, W1_rel.shape[2] // LANE)
    h = _combine_tc(x, W1_root, b1, a1, counts2, relu=True)

    t2 = _tables_tc(h, W2_rel)
    a2 = _scatter_sc(t2, pk, W2_rel.shape[2] // LANE)
    out = _combine_tc(h, W2_root, b2, a2, counts2, relu=False)
    return out


# speedup vs baseline: 1.0870x; 1.0006x over previous
import functools

import jax
import jax.numpy as jnp
from jax import lax
from jax.experimental import pallas as pl
from jax.experimental.pallas import tpu as pltpu
from jax.experimental.pallas import tpu_sc as plsc

_SC_PARAMS = pltpu.CompilerParams(use_tc_tiling_on_sc=False,
                                  needs_layout_passes=False)

N = 10000
E = 320000
NR = 8
NBINS = NR * N
NSUB = 16
NCORE = 2
LANE = 16
K = 400
RING = 5
ROWS_PER_SUB = NBINS // NSUB


def _tables_tc(h, W_rel):
    Din = h.shape[1]
    D = W_rel.shape[2]
    NCH = D // LANE
    wt = W_rel.reshape(NR, Din, NCH, LANE).transpose(2, 0, 1, 3)
    wt = wt.astype(jnp.bfloat16)
    eye8 = jnp.eye(8, dtype=jnp.bfloat16)
    W8 = (eye8[None, None, :, None, :, None]
          * wt[:, :, None, :, None, :]).reshape(NCH, NR, 8 * Din, 8 * LANE)
    x8 = h.reshape(N // 8, 8 * Din).astype(jnp.bfloat16)

    W8 = W8.reshape(NCH * NR // 4, 4, 8 * Din, 8 * LANE)

    def body(x_ref, w_ref, o_ref):
        for i in range(4):
            y = jnp.dot(x_ref[...], w_ref[0, i],
                        preferred_element_type=jnp.float32)
            o_ref[pl.ds(i * N * LANE, N * LANE)] = y.reshape(N * LANE)

    out = pl.pallas_call(
        body,
        grid=(NCH * NR // 4,),
        in_specs=[
            pl.BlockSpec((N // 8, 8 * Din), lambda q: (0, 0)),
            pl.BlockSpec((1, 4, 8 * Din, 8 * LANE), lambda q: (q, 0, 0, 0)),
        ],
        out_specs=pl.BlockSpec((4 * N * LANE,), lambda q: (q,)),
        out_shape=jax.ShapeDtypeStruct((NCH * NR * N * LANE,), jnp.float32),
    )(x8, W8)
    return out.reshape(NCH * NR * N, LANE)


def _counts_sc(pk):
    eps = E // (NCORE * NSUB)
    nb = eps // K
    mesh = plsc.VectorSubcoreMesh(core_axis_name="c", subcore_axis_name="s")

    @functools.partial(
        pl.kernel,
        out_type=jax.ShapeDtypeStruct((NCORE * NBINS,), jnp.float32),
        mesh=mesh,
        compiler_params=_SC_PARAMS,
        scratch_types=[
            pltpu.VMEM((2, K), jnp.int32),
            pltpu.VMEM((K, LANE), jnp.float32),
            pltpu.VMEM((1264, LANE), jnp.float32),
            pltpu.VMEM((ROWS_PER_SUB,), jnp.float32),
            pltpu.VMEM_SHARED((NBINS, LANE), jnp.float32),
        ],
    )
    def k(pk_hbm, out_hbm, pk_v, ones_v, piece_v, cnt_v, accum):
        core = lax.axis_index("c")
        sub = lax.axis_index("s")

        @pl.loop(0, K)
        def _(i):
            ones_v[i, :] = jnp.full((LANE,), 1.0, jnp.float32)

        @pl.loop(0, 1250)
        def _(i):
            piece_v[i, :] = jnp.zeros((LANE,), jnp.float32)

        @pl.loop(0, 4)
        def _(i):
            pltpu.sync_copy(piece_v.at[pl.ds(0, 1250)],
                            accum.at[pl.ds(sub * ROWS_PER_SUB + i * 1250, 1250)])
        plsc.subcore_barrier()

        mbase = (core * NSUB + sub) * nb

        @pl.loop(0, nb)
        def _(b):
            pltpu.sync_copy(pk_hbm.at[mbase + b], pk_v)
            pltpu.sync_copy(ones_v, accum.at[pk_v.at[1]], add=True)
        plsc.subcore_barrier()

        @pl.loop(0, 4)
        def _(i):
            start = jnp.minimum(i * 1250, ROWS_PER_SUB - 1264)
            pltpu.sync_copy(accum.at[pl.ds(sub * ROWS_PER_SUB + start, 1264)],
                            piece_v)

            @pl.loop(0, 1264 // LANE)
            def _(q):
                rows = q * LANE + lax.iota(jnp.int32, LANE)
                vals = plsc.load_gather(piece_v,
                                        [rows, jnp.zeros((LANE,), jnp.int32)])
                cnt_v[pl.ds(start + q * LANE, LANE)] = vals

        pltpu.sync_copy(
            cnt_v,
            out_hbm.at[pl.ds(core * NBINS + sub * ROWS_PER_SUB,
                             ROWS_PER_SUB)])

    return k(pk)


def _scatter_sc(table, pk, nch):
    cpc = nch // NCORE
    eps = E // NSUB
    nb = eps // K
    mesh = plsc.VectorSubcoreMesh(core_axis_name="c", subcore_axis_name="s")

    @functools.partial(
        pl.kernel,
        out_type=jax.ShapeDtypeStruct((NR, N, 8 * LANE), jnp.float32),
        mesh=mesh,
        compiler_params=_SC_PARAMS,
        scratch_types=[
            pltpu.VMEM((RING, 2, K), jnp.int32),
            pltpu.VMEM((RING, K), jnp.int32),
            pltpu.VMEM((RING, K, LANE), jnp.float32),
            pltpu.VMEM((625, LANE), jnp.float32),
            pltpu.VMEM_SHARED((NBINS, LANE), jnp.float32),
        ] + [pltpu.SemaphoreType.DMA] * (2 * RING),
    )
    def k(table_hbm, pk_hbm, out_hbm,
          pk_v, idx_v, rows_v, zero_v, accum, *sems):
        sem_g = sems[:RING]
        sem_s = sems[RING:]
        core = lax.axis_index("c")
        sub = lax.axis_index("s")
        mbase = sub * nb
        r0 = sub // 2
        n0 = (sub % 2) * ROWS_PER_SUB

        @pl.loop(0, 625)
        def _(i):
            zero_v[i, :] = jnp.zeros((LANE,), jnp.float32)

        def load_batch(j, b, off):
            pltpu.sync_copy(pk_hbm.at[mbase + b], pk_v.at[j])

            @pl.loop(0, K // LANE)
            def _(i):
                sl = pl.ds(i * LANE, LANE)
                idx_v[j, sl] = pk_v[j, 0, sl] + off

        def gather(j):
            pltpu.async_copy(table_hbm.at[idx_v.at[j]], rows_v.at[j],
                             sem_g[j])

        def wait_g(j):
            pltpu.make_async_copy(table_hbm.at[idx_v.at[j]], rows_v.at[j],
                                  sem_g[j]).wait()

        def scatter(j):
            pltpu.async_copy(rows_v.at[j], accum.at[pk_v.at[j, 1]],
                             sem_s[j], add=True)

        def wait_s(j):
            pltpu.make_async_copy(rows_v.at[j], accum.at[pk_v.at[j, 1]],
                                  sem_s[j]).wait()

        def zero_slice():
            @pl.loop(0, 8)
            def _(i):
                pltpu.sync_copy(
                    zero_v,
                    accum.at[pl.ds(sub * ROWS_PER_SUB + i * 625, 625)])

        def prologue_issue(off):
            for j in range(RING):
                load_batch(j, j, off)
                gather(j)

        def steady_and_drain(off):
            wait_g(0)
            scatter(0)

            @pl.loop(1, nb // RING)
            def _(p):
                for j in range(RING):
                    b = p * RING + j
                    wait_s(j)
                    load_batch(j, b, off)
                    gather(j)
                    jo = (j + 1) % RING
                    wait_g(jo)
                    scatter(jo)

            for j in range(1, RING):
                wait_g(j)
                scatter(j)
            for j in range(RING):
                wait_s(j)

        for kk in range(cpc):
            g = core * cpc + kk
            off = g * NBINS
            if kk == 0:
                zero_slice()
                plsc.subcore_barrier()
                prologue_issue(off)
            steady_and_drain(off)
            plsc.subcore_barrier()
            if kk + 1 < cpc:
                prologue_issue((g + 1) * NBINS)
            pltpu.sync_copy(
                accum.at[pl.ds(sub * ROWS_PER_SUB, ROWS_PER_SUB)],
                out_hbm.at[r0, pl.ds(n0, ROWS_PER_SUB),
                           pl.ds(g * LANE, LANE)])
            if kk + 1 < cpc:
                zero_slice()
                plsc.subcore_barrier()

    return k(table, pk)


def _combine_tc(h, W_root, b, acc, counts2, relu):
    D = W_root.shape[1]
    BN = 5000

    def body(x_ref, w_ref, b_ref, a_ref, c_ref, o_ref):
        r = pl.program_id(1)
        cnt = c_ref[0, 0, :, 0] + c_ref[1, 0, :, 0]
        inv = 1.0 / jnp.maximum(cnt, 1.0)
        contrib = a_ref[0, :, :D] * inv[:, None]

        @pl.when(r == 0)
        def _():
            o_ref[...] = jnp.dot(x_ref[...], w_ref[...],
                                 preferred_element_type=jnp.float32) \
                + b_ref[0] + contrib

        @pl.when(r > 0)
        def _():
            o_ref[...] += contrib

        if relu:
            @pl.when(r == NR - 1)
            def _():
                o_ref[...] = jnp.maximum(o_ref[...], 0.0)

    return pl.pallas_call(
        body,
        grid=(N // BN, NR),
        in_specs=[
            pl.BlockSpec((BN, h.shape[1]), lambda n, r: (n, 0)),
            pl.BlockSpec((h.shape[1], D), lambda n, r: (0, 0)),
            pl.BlockSpec((1, D), lambda n, r: (0, 0)),
            pl.BlockSpec((1, BN, 8 * LANE), lambda n, r: (r, n, 0)),
            pl.BlockSpec((NCORE, 1, BN, 1), lambda n, r: (0, r, n, 0)),
        ],
        out_specs=pl.BlockSpec((BN, D), lambda n, r: (n, 0)),
        out_shape=jax.ShapeDtypeStruct((N, D), jnp.float32),
    )(h, W_root, b.reshape(1, D), acc, counts2)


def kernel(x, edge_index, edge_type, W1_rel, W1_root, b1, W2_rel, W2_root, b2):
    ei = edge_index.astype(jnp.int32)
    et = edge_type.astype(jnp.int32)
    gidx = et * N + ei[0]
    sidx = et * N + ei[1]
    pk = jnp.stack([gidx.reshape(E // K, K), sidx.reshape(E // K, K)], axis=1)

    counts1d = _counts_sc(pk)
    counts2 = counts1d.reshape(NCORE, NR, N, 1)

    t1 = _tables_tc(x, W1_rel)
    a1 = _scatter_sc(t1, pk, W1_rel.shape[2] // LANE)
    h = _combine_tc(x, W1_root, b1, a1, counts2, relu=True)

    t2 = _tables_tc(h, W2_rel)
    a2 = _scatter_sc(t2, pk, W2_rel.shape[2] // LANE)
    out = _combine_tc(h, W2_root, b2, a2, counts2, relu=False)
    return out


# speedup vs baseline: 1.1905x; 1.0952x over previous
import functools

import jax
import jax.numpy as jnp
from jax import lax
from jax.experimental import pallas as pl
from jax.experimental.pallas import tpu as pltpu
from jax.experimental.pallas import tpu_sc as plsc

_SC_PARAMS = pltpu.CompilerParams(use_tc_tiling_on_sc=False,
                                  needs_layout_passes=False)

N = 10000
E = 320000
NR = 8
NBINS = NR * N
NSUB = 16
NCORE = 2
LANE = 16
K = 400
RING = 5
ROWS_PER_SUB = NBINS // NSUB


def _tables_tc(h, W_rel):
    Din = h.shape[1]
    D = W_rel.shape[2]
    NCH = D // LANE
    wt = W_rel.reshape(NR, Din, NCH, LANE).transpose(2, 0, 1, 3)
    wt = wt.astype(jnp.bfloat16).reshape(NCH * NR // 4, 4, Din, LANE)
    x8 = h.reshape(N // 8, 8 * Din).astype(jnp.bfloat16)

    def body(x_ref, w_ref, o_ref, w8_ref):
        @pl.when(pl.program_id(0) == 0)
        def _():
            w8_ref[...] = jnp.zeros((8 * Din, 8 * LANE), jnp.bfloat16)

        for i in range(4):
            for e in range(8):
                w8_ref[e * Din:(e + 1) * Din, e * LANE:(e + 1) * LANE] = \
                    w_ref[0, i]
            y = jnp.dot(x_ref[...], w8_ref[...],
                        preferred_element_type=jnp.float32)
            o_ref[pl.ds(i * N * LANE, N * LANE)] = y.reshape(N * LANE)

    out = pl.pallas_call(
        body,
        grid=(NCH * NR // 4,),
        in_specs=[
            pl.BlockSpec((N // 8, 8 * Din), lambda q: (0, 0)),
            pl.BlockSpec((1, 4, Din, LANE), lambda q: (q, 0, 0, 0)),
        ],
        out_specs=pl.BlockSpec((4 * N * LANE,), lambda q: (q,)),
        out_shape=jax.ShapeDtypeStruct((NCH * NR * N * LANE,), jnp.float32),
        scratch_shapes=[pltpu.VMEM((8 * Din, 8 * LANE), jnp.bfloat16)],
    )(x8, wt)
    return out.reshape(NCH * NR * N, LANE)


def _counts_sc(pk):
    eps = E // (NCORE * NSUB)
    nb = eps // K
    mesh = plsc.VectorSubcoreMesh(core_axis_name="c", subcore_axis_name="s")

    @functools.partial(
        pl.kernel,
        out_type=jax.ShapeDtypeStruct((NCORE * NBINS,), jnp.float32),
        mesh=mesh,
        compiler_params=_SC_PARAMS,
        scratch_types=[
            pltpu.VMEM((2, K), jnp.int32),
            pltpu.VMEM((K, LANE), jnp.float32),
            pltpu.VMEM((1264, LANE), jnp.float32),
            pltpu.VMEM((ROWS_PER_SUB,), jnp.float32),
            pltpu.VMEM_SHARED((NBINS, LANE), jnp.float32),
        ],
    )
    def k(pk_hbm, out_hbm, pk_v, ones_v, piece_v, cnt_v, accum):
        core = lax.axis_index("c")
        sub = lax.axis_index("s")

        @pl.loop(0, K)
        def _(i):
            ones_v[i, :] = jnp.full((LANE,), 1.0, jnp.float32)

        @pl.loop(0, 1250)
        def _(i):
            piece_v[i, :] = jnp.zeros((LANE,), jnp.float32)

        @pl.loop(0, 4)
        def _(i):
            pltpu.sync_copy(piece_v.at[pl.ds(0, 1250)],
                            accum.at[pl.ds(sub * ROWS_PER_SUB + i * 1250, 1250)])
        plsc.subcore_barrier()

        mbase = (core * NSUB + sub) * nb

        @pl.loop(0, nb)
        def _(b):
            pltpu.sync_copy(pk_hbm.at[mbase + b], pk_v)
            pltpu.sync_copy(ones_v, accum.at[pk_v.at[1]], add=True)
        plsc.subcore_barrier()

        @pl.loop(0, 4)
        def _(i):
            start = jnp.minimum(i * 1250, ROWS_PER_SUB - 1264)
            pltpu.sync_copy(accum.at[pl.ds(sub * ROWS_PER_SUB + start, 1264)],
                            piece_v)

            @pl.loop(0, 1264 // LANE)
            def _(q):
                rows = q * LANE + lax.iota(jnp.int32, LANE)
                vals = plsc.load_gather(piece_v,
                                        [rows, jnp.zeros((LANE,), jnp.int32)])
                cnt_v[pl.ds(start + q * LANE, LANE)] = vals

        pltpu.sync_copy(
            cnt_v,
            out_hbm.at[pl.ds(core * NBINS + sub * ROWS_PER_SUB,
                             ROWS_PER_SUB)])

    return k(pk)


def _scatter_sc(table, pk, nch):
    cpc = nch // NCORE
    eps = E // NSUB
    nb = eps // K
    mesh = plsc.VectorSubcoreMesh(core_axis_name="c", subcore_axis_name="s")

    @functools.partial(
        pl.kernel,
        out_type=jax.ShapeDtypeStruct((NR, N, 8 * LANE), jnp.float32),
        mesh=mesh,
        compiler_params=_SC_PARAMS,
        scratch_types=[
            pltpu.VMEM((RING, 2, K), jnp.int32),
            pltpu.VMEM((RING, K), jnp.int32),
            pltpu.VMEM((RING, K, LANE), jnp.float32),
            pltpu.VMEM((625, LANE), jnp.float32),
            pltpu.VMEM_SHARED((NBINS, LANE), jnp.float32),
        ] + [pltpu.SemaphoreType.DMA] * (2 * RING),
    )
    def k(table_hbm, pk_hbm, out_hbm,
          pk_v, idx_v, rows_v, zero_v, accum, *sems):
        sem_g = sems[:RING]
        sem_s = sems[RING:]
        core = lax.axis_index("c")
        sub = lax.axis_index("s")
        mbase = sub * nb
        r0 = sub // 2
        n0 = (sub % 2) * ROWS_PER_SUB

        @pl.loop(0, 625)
        def _(i):
            zero_v[i, :] = jnp.zeros((LANE,), jnp.float32)

        def load_batch(j, b, off):
            pltpu.sync_copy(pk_hbm.at[mbase + b], pk_v.at[j])

            @pl.loop(0, K // LANE)
            def _(i):
                sl = pl.ds(i * LANE, LANE)
                idx_v[j, sl] = pk_v[j, 0, sl] + off

        def gather(j):
            pltpu.async_copy(table_hbm.at[idx_v.at[j]], rows_v.at[j],
                             sem_g[j])

        def wait_g(j):
            pltpu.make_async_copy(table_hbm.at[idx_v.at[j]], rows_v.at[j],
                                  sem_g[j]).wait()

        def scatter(j):
            pltpu.async_copy(rows_v.at[j], accum.at[pk_v.at[j, 1]],
                             sem_s[j], add=True)

        def wait_s(j):
            pltpu.make_async_copy(rows_v.at[j], accum.at[pk_v.at[j, 1]],
                                  sem_s[j]).wait()

        def zero_slice():
            @pl.loop(0, 8)
            def _(i):
                pltpu.sync_copy(
                    zero_v,
                    accum.at[pl.ds(sub * ROWS_PER_SUB + i * 625, 625)])

        def prologue_issue(off):
            for j in range(RING):
                load_batch(j, j, off)
                gather(j)

        def steady_and_drain(off):
            wait_g(0)
            scatter(0)

            @pl.loop(1, nb // RING)
            def _(p):
                for j in range(RING):
                    b = p * RING + j
                    wait_s(j)
                    load_batch(j, b, off)
                    gather(j)
                    jo = (j + 1) % RING
                    wait_g(jo)
                    scatter(jo)

            for j in range(1, RING):
                wait_g(j)
                scatter(j)
            for j in range(RING):
                wait_s(j)

        for kk in range(cpc):
            g = core * cpc + kk
            off = g * NBINS
            if kk == 0:
                zero_slice()
                plsc.subcore_barrier()
                prologue_issue(off)
            steady_and_drain(off)
            plsc.subcore_barrier()
            if kk + 1 < cpc:
                prologue_issue((g + 1) * NBINS)
            pltpu.sync_copy(
                accum.at[pl.ds(sub * ROWS_PER_SUB, ROWS_PER_SUB)],
                out_hbm.at[r0, pl.ds(n0, ROWS_PER_SUB),
                           pl.ds(g * LANE, LANE)])
            if kk + 1 < cpc:
                zero_slice()
                plsc.subcore_barrier()

    return k(table, pk)


def _combine_tc(h, W_root, b, acc, counts2, relu):
    D = W_root.shape[1]
    BN = 5000

    def body(x_ref, w_ref, b_ref, a_ref, c_ref, o_ref):
        r = pl.program_id(1)
        cnt = c_ref[0, 0, :, 0] + c_ref[1, 0, :, 0]
        inv = 1.0 / jnp.maximum(cnt, 1.0)
        contrib = a_ref[0, :, :D] * inv[:, None]

        @pl.when(r == 0)
        def _():
            o_ref[...] = jnp.dot(x_ref[...], w_ref[...],
                                 preferred_element_type=jnp.float32) \
                + b_ref[0] + contrib

        @pl.when(r > 0)
        def _():
            o_ref[...] += contrib

        if relu:
            @pl.when(r == NR - 1)
            def _():
                o_ref[...] = jnp.maximum(o_ref[...], 0.0)

    return pl.pallas_call(
        body,
        grid=(N // BN, NR),
        in_specs=[
            pl.BlockSpec((BN, h.shape[1]), lambda n, r: (n, 0)),
            pl.BlockSpec((h.shape[1], D), lambda n, r: (0, 0)),
            pl.BlockSpec((1, D), lambda n, r: (0, 0)),
            pl.BlockSpec((1, BN, 8 * LANE), lambda n, r: (r, n, 0)),
            pl.BlockSpec((NCORE, 1, BN, 1), lambda n, r: (0, r, n, 0)),
        ],
        out_specs=pl.BlockSpec((BN, D), lambda n, r: (n, 0)),
        out_shape=jax.ShapeDtypeStruct((N, D), jnp.float32),
    )(h, W_root, b.reshape(1, D), acc, counts2)


def kernel(x, edge_index, edge_type, W1_rel, W1_root, b1, W2_rel, W2_root, b2):
    ei = edge_index.astype(jnp.int32)
    et = edge_type.astype(jnp.int32)
    gidx = et * N + ei[0]
    sidx = et * N + ei[1]
    pk = jnp.stack([gidx.reshape(E // K, K), sidx.reshape(E // K, K)], axis=1)

    counts1d = _counts_sc(pk)
    counts2 = counts1d.reshape(NCORE, NR, N, 1)

    t1 = _tables_tc(x, W1_rel)
    a1 = _scatter_sc(t1, pk, W1_rel.shape[2] // LANE)
    h = _combine_tc(x, W1_root, b1, a1, counts2, relu=True)

    t2 = _tables_tc(h, W2_rel)
    a2 = _scatter_sc(t2, pk, W2_rel.shape[2] // LANE)
    out = _combine_tc(h, W2_root, b2, a2, counts2, relu=False)
    return out
